# baseline (speedup 1.0000x reference)
import functools

import jax
import jax.numpy as jnp
from jax.experimental import pallas as pl
from jax.experimental.pallas import tpu as pltpu
from jax.experimental.pallas import tpu_sc as plsc

NUM_CORES = 2
NUM_SUBCORES = 16
CHUNK = 800
BLOCK_M = 2048
REPACK_BK = 16128
QTR = REPACK_BK // 4


def _repack_tc(tableT):
    K, V = tableT.shape
    H = K // 2
    n_blocks = pl.cdiv(V, REPACK_BK)

    def rp_kernel(x_ref, eye_ref, out_ref):
        for q in range(4):
            xb = x_ref[:, q * QTR:(q + 1) * QTR].astype(jnp.bfloat16)
            if q % 2 == 0:
                y = xb.T.astype(jnp.float32)
            else:
                y = jax.lax.dot_general(
                    xb, eye_ref[...], (((0,), (0,)), ((), ())),
                    preferred_element_type=jnp.float32)
            u = jax.lax.bitcast_convert_type(y, jnp.uint32)
            w = (u[:, :H] >> 16) | u[:, H:]
            out_ref[:, q * H:(q + 1) * H] = (
                jax.lax.bitcast_convert_type(w, jnp.float32))

    eye = jnp.eye(K, dtype=jnp.bfloat16)
    return pl.pallas_call(
        rp_kernel,
        out_shape=jax.ShapeDtypeStruct((n_blocks * QTR, 2 * K), jnp.float32),
        grid=(n_blocks,),
        in_specs=[
            pl.BlockSpec((K, REPACK_BK), lambda i: (0, i)),
            pl.BlockSpec((K, K), lambda i: (0, 0)),
        ],
        out_specs=pl.BlockSpec((QTR, 2 * K), lambda i: (i, 0)),
    )(tableT, eye)


def _gather_sc(table_lin, flat_ids):
    n = flat_ids.shape[0]
    width = table_lin.shape[1]
    nw = NUM_CORES * NUM_SUBCORES
    b_per_w = n // nw
    n_chunks = b_per_w // CHUNK

    mesh = plsc.VectorSubcoreMesh(core_axis_name="c", subcore_axis_name="s")

    @functools.partial(
        pl.kernel, mesh=mesh,
        out_type=jax.ShapeDtypeStruct((n, width), jnp.float32),
        scratch_types=[
            pltpu.VMEM((CHUNK,), jnp.int32),
            pltpu.VMEM((CHUNK, width), jnp.float32),
            pltpu.SemaphoreType.DMA,
        ],
        compiler_params=pltpu.CompilerParams(use_tc_tiling_on_sc=False),
    )
    def gather_kernel(table_hbm, idx_hbm, out_hbm, idx_v, rows_v, sem):
        wid = jax.lax.axis_index("s") * NUM_CORES + jax.lax.axis_index("c")
        base_w = wid * b_per_w

        @pl.loop(0, n_chunks)
        def _(g):
            base = base_w + g * CHUNK
            pltpu.sync_copy(idx_hbm.at[pl.ds(base, CHUNK)], idx_v)
            pltpu.async_copy(table_hbm.at[idx_v], rows_v, sem).wait()
            pltpu.sync_copy(rows_v, out_hbm.at[pl.ds(base, CHUNK)])

    return gather_kernel(table_lin, flat_ids)


def _matmul_tc(emb4, W, b):
    M4 = emb4.shape[0]
    K, N = W.shape
    H = K // 2
    BM4 = BLOCK_M // 4

    def mm_kernel(emb_ref, w_ref, b_ref, out_ref):
        u = jax.lax.bitcast_convert_type(emb_ref[...], jnp.uint32)
        lo = jax.lax.bitcast_convert_type(u << 16,
                                          jnp.float32).astype(jnp.bfloat16)
        hi = jax.lax.bitcast_convert_type(u & jnp.uint32(0xFFFF0000),
                                          jnp.float32).astype(jnp.bfloat16)
        wb = w_ref[...].astype(jnp.bfloat16)
        bias = b_ref[...]
        for s in range(4):
            acc = jnp.dot(lo[:, s * H:(s + 1) * H], wb[:H],
                          preferred_element_type=jnp.float32)
            acc += jnp.dot(hi[:, s * H:(s + 1) * H], wb[H:],
                           preferred_element_type=jnp.float32)
            out_ref[s * BM4:(s + 1) * BM4] = acc + bias

    return pl.pallas_call(
        mm_kernel,
        out_shape=jax.ShapeDtypeStruct((4 * M4, N), jnp.float32),
        grid=(M4 // BM4,),
        in_specs=[
            pl.BlockSpec((BM4, 4 * H), lambda i: (i, 0)),
            pl.BlockSpec((K, N), lambda i: (0, 0)),
            pl.BlockSpec((1, N), lambda i: (0, 0)),
        ],
        out_specs=pl.BlockSpec((BLOCK_M, N), lambda i: (i, 0)),
    )(emb4, W, b.reshape(1, N))


def kernel(token_ids, table, W, b):
    B, L = token_ids.shape
    N = W.shape[1]
    idx_lmajor = token_ids.T.reshape(-1)
    p = idx_lmajor % REPACK_BK
    idx_perm = (idx_lmajor - p) + 4 * (p % QTR) + p // QTR
    bm4 = BLOCK_M // 4
    idx_perm = idx_perm.reshape(-1, 4, bm4).swapaxes(1, 2).reshape(-1)
    packed = _repack_tc(table.T)
    table_lin = packed.reshape(packed.shape[0] * 4, table.shape[1] // 2)
    emb = _gather_sc(table_lin, idx_perm)
    emb4 = emb.reshape(emb.shape[0] // 4, 4 * emb.shape[1])
    out_lmajor = _matmul_tc(emb4, W, b)
    return out_lmajor.reshape(L, B, N).transpose(1, 0, 2)


# speedup vs baseline: 1.1495x; 1.1495x over previous
import functools

import jax
import jax.numpy as jnp
from jax.experimental import pallas as pl
from jax.experimental.pallas import tpu as pltpu
from jax.experimental.pallas import tpu_sc as plsc

NUM_CORES = 2
NUM_SUBCORES = 16
CHUNK = 800
BLOCK_M = 2048
REPACK_BK = 16128
QTR = REPACK_BK // 4


def _repack_tc(tableT):
    K, V = tableT.shape
    H = K // 2
    n_blocks = pl.cdiv(V, REPACK_BK)

    def rp_kernel(x_ref, out_ref):
        xb = x_ref[...].astype(jnp.bfloat16).astype(jnp.float32)
        u = jax.lax.bitcast_convert_type(xb, jnp.uint32)
        w = (u[:H] >> 16) | u[H:]
        z = jax.lax.bitcast_convert_type(w, jnp.float32).T
        out_ref[:, 0:H] = z[0:QTR]
        out_ref[:, H:2 * H] = z[QTR:2 * QTR]
        out_ref[:, 2 * H:3 * H] = z[2 * QTR:3 * QTR]
        out_ref[:, 3 * H:] = z[3 * QTR:]

    return pl.pallas_call(
        rp_kernel,
        out_shape=jax.ShapeDtypeStruct((n_blocks * QTR, 2 * K), jnp.float32),
        grid=(n_blocks,),
        in_specs=[pl.BlockSpec((K, REPACK_BK), lambda i: (0, i))],
        out_specs=pl.BlockSpec((QTR, 2 * K), lambda i: (i, 0)),
    )(tableT)


def _gather_sc(table_lin, flat_ids):
    n = flat_ids.shape[0]
    width = table_lin.shape[1]
    nw = NUM_CORES * NUM_SUBCORES
    b_per_w = n // nw
    n_chunks = b_per_w // CHUNK

    mesh = plsc.VectorSubcoreMesh(core_axis_name="c", subcore_axis_name="s")

    @functools.partial(
        pl.kernel, mesh=mesh,
        out_type=jax.ShapeDtypeStruct((n, width), jnp.float32),
        scratch_types=[
            pltpu.VMEM((CHUNK,), jnp.int32),
            pltpu.VMEM((CHUNK, width), jnp.float32),
            pltpu.SemaphoreType.DMA,
        ],
        compiler_params=pltpu.CompilerParams(use_tc_tiling_on_sc=False),
    )
    def gather_kernel(table_hbm, idx_hbm, out_hbm, idx_v, rows_v, sem):
        wid = jax.lax.axis_index("s") * NUM_CORES + jax.lax.axis_index("c")
        base_w = wid * b_per_w

        @pl.loop(0, n_chunks)
        def _(g):
            base = base_w + g * CHUNK
            pltpu.sync_copy(idx_hbm.at[pl.ds(base, CHUNK)], idx_v)
            pltpu.async_copy(table_hbm.at[idx_v], rows_v, sem).wait()
            pltpu.sync_copy(rows_v, out_hbm.at[pl.ds(base, CHUNK)])

    return gather_kernel(table_lin, flat_ids)


def _matmul_tc(emb4, W, b):
    M4 = emb4.shape[0]
    K, N = W.shape
    H = K // 2
    BM4 = BLOCK_M // 4

    def mm_kernel(emb_ref, w_ref, b_ref, out_ref):
        u = jax.lax.bitcast_convert_type(emb_ref[...], jnp.uint32)
        lo = jax.lax.bitcast_convert_type(u << 16,
                                          jnp.float32).astype(jnp.bfloat16)
        hi = jax.lax.bitcast_convert_type(u & jnp.uint32(0xFFFF0000),
                                          jnp.float32).astype(jnp.bfloat16)
        wb = w_ref[...].astype(jnp.bfloat16)
        bias = b_ref[...]
        for s in range(4):
            acc = jnp.dot(lo[:, s * H:(s + 1) * H], wb[:H],
                          preferred_element_type=jnp.float32)
            acc += jnp.dot(hi[:, s * H:(s + 1) * H], wb[H:],
                           preferred_element_type=jnp.float32)
            out_ref[s * BM4:(s + 1) * BM4] = acc + bias

    return pl.pallas_call(
        mm_kernel,
        out_shape=jax.ShapeDtypeStruct((4 * M4, N), jnp.float32),
        grid=(M4 // BM4,),
        in_specs=[
            pl.BlockSpec((BM4, 4 * H), lambda i: (i, 0)),
            pl.BlockSpec((K, N), lambda i: (0, 0)),
            pl.BlockSpec((1, N), lambda i: (0, 0)),
        ],
        out_specs=pl.BlockSpec((BLOCK_M, N), lambda i: (i, 0)),
    )(emb4, W, b.reshape(1, N))


def kernel(token_ids, table, W, b):
    B, L = token_ids.shape
    N = W.shape[1]
    idx_lmajor = token_ids.T.reshape(-1)
    p = idx_lmajor % REPACK_BK
    idx_perm = (idx_lmajor - p) + 4 * (p % QTR) + p // QTR
    bm4 = BLOCK_M // 4
    idx_perm = idx_perm.reshape(-1, 4, bm4).swapaxes(1, 2).reshape(-1)
    packed = _repack_tc(table.T)
    table_lin = packed.reshape(packed.shape[0] * 4, table.shape[1] // 2)
    emb = _gather_sc(table_lin, idx_perm)
    emb4 = emb.reshape(emb.shape[0] // 4, 4 * emb.shape[1])
    out_lmajor = _matmul_tc(emb4, W, b)
    return out_lmajor.reshape(L, B, N).transpose(1, 0, 2)


# speedup vs baseline: 1.2257x; 1.0663x over previous
import functools

import jax
import jax.numpy as jnp
from jax.experimental import pallas as pl
from jax.experimental.pallas import tpu as pltpu
from jax.experimental.pallas import tpu_sc as plsc

NUM_CORES = 2
NUM_SUBCORES = 16
CHUNK = 800
BLOCK_M = 4096
REPACK_BK = 32256
QTR = REPACK_BK // 4


def _repack_tc(tableT):
    K, V = tableT.shape
    H = K // 2
    n_blocks = pl.cdiv(V, REPACK_BK)

    def rp_kernel(x_ref, out_ref):
        xb = x_ref[...].astype(jnp.bfloat16).astype(jnp.float32)
        u = jax.lax.bitcast_convert_type(xb, jnp.uint32)
        w = (u[:H] >> 16) | u[H:]
        z = jax.lax.bitcast_convert_type(w, jnp.float32).T
        out_ref[:, 0:H] = z[0:QTR]
        out_ref[:, H:2 * H] = z[QTR:2 * QTR]
        out_ref[:, 2 * H:3 * H] = z[2 * QTR:3 * QTR]
        out_ref[:, 3 * H:] = z[3 * QTR:]

    return pl.pallas_call(
        rp_kernel,
        out_shape=jax.ShapeDtypeStruct((n_blocks * QTR, 2 * K), jnp.float32),
        grid=(n_blocks,),
        in_specs=[pl.BlockSpec((K, REPACK_BK), lambda i: (0, i))],
        out_specs=pl.BlockSpec((QTR, 2 * K), lambda i: (i, 0)),
    )(tableT)


def _gather_sc(table_lin, flat_ids):
    n = flat_ids.shape[0]
    width = table_lin.shape[1]
    nw = NUM_CORES * NUM_SUBCORES
    b_per_w = n // nw
    n_chunks = b_per_w // CHUNK

    mesh = plsc.VectorSubcoreMesh(core_axis_name="c", subcore_axis_name="s")

    @functools.partial(
        pl.kernel, mesh=mesh,
        out_type=jax.ShapeDtypeStruct((n, width), jnp.float32),
        scratch_types=[
            pltpu.VMEM((CHUNK,), jnp.int32),
            pltpu.VMEM((CHUNK, width), jnp.float32),
            pltpu.SemaphoreType.DMA,
        ],
        compiler_params=pltpu.CompilerParams(use_tc_tiling_on_sc=False),
    )
    def gather_kernel(table_hbm, idx_hbm, out_hbm, idx_v, rows_v, sem):
        wid = jax.lax.axis_index("s") * NUM_CORES + jax.lax.axis_index("c")
        base_w = wid * b_per_w

        @pl.loop(0, n_chunks)
        def _(g):
            base = base_w + g * CHUNK
            pltpu.sync_copy(idx_hbm.at[pl.ds(base, CHUNK)], idx_v)
            pltpu.async_copy(table_hbm.at[idx_v], rows_v, sem).wait()
            pltpu.sync_copy(rows_v, out_hbm.at[pl.ds(base, CHUNK)])

    return gather_kernel(table_lin, flat_ids)


def _matmul_tc(emb4, W, b):
    M4 = emb4.shape[0]
    K, N = W.shape
    H = K // 2
    BM4 = BLOCK_M // 4

    def mm_kernel(emb_ref, w_ref, b_ref, out_ref):
        u = jax.lax.bitcast_convert_type(emb_ref[...], jnp.uint32)
        lo = jax.lax.bitcast_convert_type(u << 16,
                                          jnp.float32).astype(jnp.bfloat16)
        hi = jax.lax.bitcast_convert_type(u & jnp.uint32(0xFFFF0000),
                                          jnp.float32).astype(jnp.bfloat16)
        wb = w_ref[...].astype(jnp.bfloat16)
        bias = b_ref[...]
        for s in range(4):
            acc = jnp.dot(lo[:, s * H:(s + 1) * H], wb[:H],
                          preferred_element_type=jnp.float32)
            acc += jnp.dot(hi[:, s * H:(s + 1) * H], wb[H:],
                           preferred_element_type=jnp.float32)
            out_ref[s * BM4:(s + 1) * BM4] = acc + bias

    return pl.pallas_call(
        mm_kernel,
        out_shape=jax.ShapeDtypeStruct((4 * M4, N), jnp.float32),
        grid=(M4 // BM4,),
        in_specs=[
            pl.BlockSpec((BM4, 4 * H), lambda i: (i, 0)),
            pl.BlockSpec((K, N), lambda i: (0, 0)),
            pl.BlockSpec((1, N), lambda i: (0, 0)),
        ],
        out_specs=pl.BlockSpec((BLOCK_M, N), lambda i: (i, 0)),
    )(emb4, W, b.reshape(1, N))


def kernel(token_ids, table, W, b):
    B, L = token_ids.shape
    N = W.shape[1]
    idx_lmajor = token_ids.T.reshape(-1)
    p = idx_lmajor % REPACK_BK
    idx_perm = (idx_lmajor - p) + 4 * (p % QTR) + p // QTR
    bm4 = BLOCK_M // 4
    idx_perm = idx_perm.reshape(-1, 4, bm4).swapaxes(1, 2).reshape(-1)
    packed = _repack_tc(table.T)
    table_lin = packed.reshape(packed.shape[0] * 4, table.shape[1] // 2)
    emb = _gather_sc(table_lin, idx_perm)
    emb4 = emb.reshape(emb.shape[0] // 4, 4 * emb.shape[1])
    out_lmajor = _matmul_tc(emb4, W, b)
    return out_lmajor.reshape(L, B, N).transpose(1, 0, 2)
